# baseline (speedup 1.0000x reference)
import functools

import jax
import jax.numpy as jnp
from jax import lax
from jax.experimental import pallas as pl
from jax.experimental.pallas import tpu as pltpu
from jax.experimental.pallas import tpu_sc as plsc

NC = 2
NS = 16
NW = NC * NS

C = 128
RB = 2048


def _gather_body(u_hbm, i_hbm, p_hbm, q_hbm, pu_hbm, qi_hbm,
                 idxu_v, idxi_v, bufs, sem_g, sem_w):
    b = u_hbm.shape[0]
    b_per_w = b // NW
    n_chunks = b_per_w // C
    n_t = 2 * n_chunks
    n_buf = len(bufs)

    wid = lax.axis_index("s") * NC + lax.axis_index("c")
    wbase = wid * b_per_w

    pltpu.sync_copy(u_hbm.at[pl.ds(wbase, b_per_w)], idxu_v)
    pltpu.sync_copy(i_hbm.at[pl.ds(wbase, b_per_w)], idxi_v)

    def plan(t):
        if t < n_chunks:
            return p_hbm, idxu_v, pu_hbm, t
        return q_hbm, idxi_v, qi_hbm, t - n_chunks

    g_h = [None] * n_t
    w_h = [None] * n_t
    for t in range(n_t):
        if t >= n_buf:
            w_h[t - n_buf].wait()
        tab, idxv, _, c = plan(t)
        g_h[t] = pltpu.async_copy(tab.at[idxv.at[pl.ds(c * C, C)]],
                                  bufs[t % n_buf], sem_g)
        if t >= 1:
            g_h[t - 1].wait()
            _, _, out, cp = plan(t - 1)
            w_h[t - 1] = pltpu.async_copy(
                bufs[(t - 1) % n_buf], out.at[pl.ds(wbase + cp * C, C)],
                sem_w)
    g_h[n_t - 1].wait()
    _, _, out, cp = plan(n_t - 1)
    w_h[n_t - 1] = pltpu.async_copy(
        bufs[(n_t - 1) % n_buf], out.at[pl.ds(wbase + cp * C, C)], sem_w)
    for t in range(n_t - n_buf, n_t):
        w_h[t].wait()


def _gather_sc(u32, i32, p, q):
    b = u32.shape[0]
    k_dim = p.shape[1]
    mesh = plsc.VectorSubcoreMesh(core_axis_name="c", subcore_axis_name="s",
                                  num_cores=NC, num_subcores=NS)
    return pl.kernel(
        _gather_body,
        out_type=[
            jax.ShapeDtypeStruct((b, k_dim), jnp.float32),
            jax.ShapeDtypeStruct((b, k_dim), jnp.float32),
        ],
        mesh=mesh,
        compiler_params=pltpu.CompilerParams(needs_layout_passes=False),
        scratch_types=[
            pltpu.VMEM((b // NW,), jnp.int32),
            pltpu.VMEM((b // NW,), jnp.int32),
            [pltpu.VMEM((C, k_dim), jnp.float32) for _ in range(4)],
            pltpu.SemaphoreType.DMA,
            pltpu.SemaphoreType.DMA,
        ],
    )(u32, i32, p, q)


def _dense_body(pu_ref, qi_ref, rst_ref, th_ref, bias_ref, rhat_ref, alt_ref):
    k_dim = pu_ref.shape[1]
    th = th_ref[...]
    dn = (((1,), (1,)), ((), ()))
    lt = lax.dot_general(th[:, :k_dim], pu_ref[...], dn,
                         preferred_element_type=jnp.float32)
    lt += lax.dot_general(th[:, k_dim:], qi_ref[...], dn,
                          preferred_element_type=jnp.float32)
    mx = jnp.max(lt, axis=0, keepdims=True)
    e = jnp.exp(lt - mx)
    al = e / jnp.sum(e, axis=0, keepdims=True)
    alt_ref[...] = al
    rhat_ref[...] = (jnp.sum(al * rst_ref[...], axis=0, keepdims=True)
                     + bias_ref[0])


def _dense_tc(pu, qi, rst, theta, bias, blk_off):
    b, k_dim = pu.shape
    num_models = rst.shape[0]
    grid = (b // RB,)
    return pl.pallas_call(
        _dense_body,
        grid=grid,
        in_specs=[
            pl.BlockSpec((RB, k_dim), lambda i: (i, 0)),
            pl.BlockSpec((RB, k_dim), lambda i: (i, 0)),
            pl.BlockSpec((num_models, RB), lambda i: (0, i + blk_off)),
            pl.BlockSpec((num_models, 2 * k_dim), lambda i: (0, 0)),
            pl.BlockSpec(memory_space=pltpu.SMEM),
        ],
        out_specs=[
            pl.BlockSpec((1, RB), lambda i: (0, i)),
            pl.BlockSpec((num_models, RB), lambda i: (0, i)),
        ],
        out_shape=[
            jax.ShapeDtypeStruct((1, b), jnp.float32),
            jax.ShapeDtypeStruct((num_models, b), jnp.float32),
        ],
    )(pu, qi, rst, theta, bias)


@jax.jit
def _pla(u_idx, i_idx, r_s, p, q, theta, bias):
    b = u_idx.shape[0]
    h = b // 2
    u32 = u_idx.astype(jnp.int32)
    i32 = i_idx.astype(jnp.int32)
    rst = r_s.T
    pu1, qi1 = _gather_sc(u32[:h], i32[:h], p, q)
    pu2, qi2 = _gather_sc(u32[h:], i32[h:], p, q)
    rhat1, alt1 = _dense_tc(pu1, qi1, rst, theta, bias, 0)
    rhat2, alt2 = _dense_tc(pu2, qi2, rst, theta, bias, h // RB)
    rhat = jnp.concatenate([rhat1, rhat2], axis=1)
    alt = jnp.concatenate([alt1, alt2], axis=1)
    return rhat.reshape(b), alt.T


def kernel(u_idx, i_idx, r_s, P, Q, theta, bias):
    r_hat, alphas = _pla(u_idx, i_idx, r_s, P, Q, theta, bias)
    return (r_hat, alphas, r_s)


# speedup vs baseline: 1.1331x; 1.1331x over previous
import functools

import jax
import jax.numpy as jnp
from jax import lax
from jax.experimental import pallas as pl
from jax.experimental.pallas import tpu as pltpu
from jax.experimental.pallas import tpu_sc as plsc

NC = 2
NS = 16
NW = NC * NS

C = 64
N_BUF = 8
RB = 4096


def _gather_body(u_hbm, i_hbm, p_hbm, q_hbm, pu_hbm, qi_hbm,
                 idxu_v, idxi_v, bufs, sem_g, sem_w):
    b = u_hbm.shape[0]
    b_per_w = b // NW
    n_chunks = b_per_w // C
    n_t = 2 * n_chunks
    n_buf = len(bufs)

    wid = lax.axis_index("s") * NC + lax.axis_index("c")
    wbase = wid * b_per_w

    pltpu.sync_copy(u_hbm.at[pl.ds(wbase, b_per_w)], idxu_v)
    pltpu.sync_copy(i_hbm.at[pl.ds(wbase, b_per_w)], idxi_v)

    def plan(t):
        if t < n_chunks:
            return p_hbm, idxu_v, pu_hbm, t
        return q_hbm, idxi_v, qi_hbm, t - n_chunks

    g_h = [None] * n_t
    w_h = [None] * n_t
    for t in range(n_t):
        if t >= n_buf:
            w_h[t - n_buf].wait()
        tab, idxv, _, c = plan(t)
        g_h[t] = pltpu.async_copy(tab.at[idxv.at[pl.ds(c * C, C)]],
                                  bufs[t % n_buf], sem_g)
        if t >= 1:
            g_h[t - 1].wait()
            _, _, out, cp = plan(t - 1)
            w_h[t - 1] = pltpu.async_copy(
                bufs[(t - 1) % n_buf], out.at[pl.ds(wbase + cp * C, C)],
                sem_w)
    g_h[n_t - 1].wait()
    _, _, out, cp = plan(n_t - 1)
    w_h[n_t - 1] = pltpu.async_copy(
        bufs[(n_t - 1) % n_buf], out.at[pl.ds(wbase + cp * C, C)], sem_w)
    for t in range(n_t - n_buf, n_t):
        w_h[t].wait()


def _gather_sc(u32, i32, p, q):
    b = u32.shape[0]
    k_dim = p.shape[1]
    mesh = plsc.VectorSubcoreMesh(core_axis_name="c", subcore_axis_name="s",
                                  num_cores=NC, num_subcores=NS)
    return pl.kernel(
        _gather_body,
        out_type=[
            jax.ShapeDtypeStruct((b, k_dim), jnp.float32),
            jax.ShapeDtypeStruct((b, k_dim), jnp.float32),
        ],
        mesh=mesh,
        compiler_params=pltpu.CompilerParams(needs_layout_passes=False),
        scratch_types=[
            pltpu.VMEM((b // NW,), jnp.int32),
            pltpu.VMEM((b // NW,), jnp.int32),
            [pltpu.VMEM((C, k_dim), jnp.float32) for _ in range(N_BUF)],
            pltpu.SemaphoreType.DMA,
            pltpu.SemaphoreType.DMA,
        ],
    )(u32, i32, p, q)


def _dense_body(pu_ref, qi_ref, rst_ref, th_ref, bias_ref, rhat_ref, alt_ref):
    k_dim = pu_ref.shape[1]
    th = th_ref[...]
    dn = (((1,), (1,)), ((), ()))
    lt = lax.dot_general(th[:, :k_dim], pu_ref[...], dn,
                         preferred_element_type=jnp.float32)
    lt += lax.dot_general(th[:, k_dim:], qi_ref[...], dn,
                          preferred_element_type=jnp.float32)
    mx = jnp.max(lt, axis=0, keepdims=True)
    e = jnp.exp(lt - mx)
    al = e / jnp.sum(e, axis=0, keepdims=True)
    alt_ref[...] = al
    rhat_ref[...] = (jnp.sum(al * rst_ref[...], axis=0, keepdims=True)
                     + bias_ref[0])


def _dense_tc(pu, qi, rst, theta, bias):
    b, k_dim = pu.shape
    num_models = rst.shape[0]
    grid = (b // RB,)
    return pl.pallas_call(
        _dense_body,
        grid=grid,
        in_specs=[
            pl.BlockSpec((RB, k_dim), lambda i: (i, 0)),
            pl.BlockSpec((RB, k_dim), lambda i: (i, 0)),
            pl.BlockSpec((num_models, RB), lambda i: (0, i)),
            pl.BlockSpec((num_models, 2 * k_dim), lambda i: (0, 0)),
            pl.BlockSpec(memory_space=pltpu.SMEM),
        ],
        out_specs=[
            pl.BlockSpec((1, RB), lambda i: (0, i)),
            pl.BlockSpec((num_models, RB), lambda i: (0, i)),
        ],
        out_shape=[
            jax.ShapeDtypeStruct((1, b), jnp.float32),
            jax.ShapeDtypeStruct((num_models, b), jnp.float32),
        ],
    )(pu, qi, rst, theta, bias)


@jax.jit
def _pla(u_idx, i_idx, r_s, p, q, theta, bias):
    u32 = u_idx.astype(jnp.int32)
    i32 = i_idx.astype(jnp.int32)
    pu, qi = _gather_sc(u32, i32, p, q)
    rhat2, alt = _dense_tc(pu, qi, r_s.T, theta, bias)
    return rhat2.reshape(r_s.shape[0]), alt.T


def kernel(u_idx, i_idx, r_s, P, Q, theta, bias):
    r_hat, alphas = _pla(u_idx, i_idx, r_s, P, Q, theta, bias)
    return (r_hat, alphas, r_s)


# speedup vs baseline: 1.1391x; 1.0053x over previous
import functools

import jax
import jax.numpy as jnp
from jax import lax
from jax.experimental import pallas as pl
from jax.experimental.pallas import tpu as pltpu
from jax.experimental.pallas import tpu_sc as plsc

NC = 2
NS = 16
NW = NC * NS

C = 64
N_BUF = 8
RB = 4096


def _gather_body(u_hbm, i_hbm, p_hbm, q_hbm, pu_hbm, qi_hbm,
                 idxu_v, idxi_v, bufs, sem_g, sem_w):
    b = u_hbm.shape[0]
    b_per_w = b // NW
    n_chunks = b_per_w // C
    n_t = 2 * n_chunks
    n_buf = len(bufs)

    wid = lax.axis_index("s") * NC + lax.axis_index("c")
    wbase = wid * b_per_w

    pltpu.sync_copy(u_hbm.at[pl.ds(wbase, b_per_w)], idxu_v)
    pltpu.sync_copy(i_hbm.at[pl.ds(wbase, b_per_w)], idxi_v)

    def plan(t):
        if t < n_chunks:
            return p_hbm, idxu_v, pu_hbm, t
        return q_hbm, idxi_v, qi_hbm, t - n_chunks

    g_h = [None] * n_t
    w_h = [None] * n_t
    for t in range(n_t):
        if t >= n_buf:
            w_h[t - n_buf].wait()
        tab, idxv, _, c = plan(t)
        g_h[t] = pltpu.async_copy(tab.at[idxv.at[pl.ds(c * C, C)]],
                                  bufs[t % n_buf], sem_g)
        if t >= 1:
            g_h[t - 1].wait()
            _, _, out, cp = plan(t - 1)
            w_h[t - 1] = pltpu.async_copy(
                bufs[(t - 1) % n_buf], out.at[pl.ds(wbase + cp * C, C)],
                sem_w)
    g_h[n_t - 1].wait()
    _, _, out, cp = plan(n_t - 1)
    w_h[n_t - 1] = pltpu.async_copy(
        bufs[(n_t - 1) % n_buf], out.at[pl.ds(wbase + cp * C, C)], sem_w)
    for t in range(n_t - n_buf, n_t):
        w_h[t].wait()


def _gather_sc(u32, i32, p, q):
    b = u32.shape[0]
    k_dim = p.shape[1]
    mesh = plsc.VectorSubcoreMesh(core_axis_name="c", subcore_axis_name="s",
                                  num_cores=NC, num_subcores=NS)
    return pl.kernel(
        _gather_body,
        out_type=[
            jax.ShapeDtypeStruct((b, k_dim), jnp.float32),
            jax.ShapeDtypeStruct((b, k_dim), jnp.float32),
        ],
        mesh=mesh,
        compiler_params=pltpu.CompilerParams(needs_layout_passes=False,
                                             skip_device_barrier=True),
        scratch_types=[
            pltpu.VMEM((b // NW,), jnp.int32),
            pltpu.VMEM((b // NW,), jnp.int32),
            [pltpu.VMEM((C, k_dim), jnp.float32) for _ in range(N_BUF)],
            pltpu.SemaphoreType.DMA,
            pltpu.SemaphoreType.DMA,
        ],
    )(u32, i32, p, q)


def _dense_body(pu_ref, qi_ref, rst_ref, th_ref, bias_ref, rhat_ref, alt_ref):
    k_dim = pu_ref.shape[1]
    th = th_ref[...]
    dn = (((1,), (1,)), ((), ()))
    lt = lax.dot_general(th[:, :k_dim], pu_ref[...], dn,
                         preferred_element_type=jnp.float32)
    lt += lax.dot_general(th[:, k_dim:], qi_ref[...], dn,
                          preferred_element_type=jnp.float32)
    mx = jnp.max(lt, axis=0, keepdims=True)
    e = jnp.exp(lt - mx)
    al = e / jnp.sum(e, axis=0, keepdims=True)
    alt_ref[...] = al
    rhat_ref[...] = (jnp.sum(al * rst_ref[...], axis=0, keepdims=True)
                     + bias_ref[0])


def _dense_tc(pu, qi, rst, theta, bias):
    b, k_dim = pu.shape
    num_models = rst.shape[0]
    grid = (b // RB,)
    return pl.pallas_call(
        _dense_body,
        grid=grid,
        in_specs=[
            pl.BlockSpec((RB, k_dim), lambda i: (i, 0)),
            pl.BlockSpec((RB, k_dim), lambda i: (i, 0)),
            pl.BlockSpec((num_models, RB), lambda i: (0, i)),
            pl.BlockSpec((num_models, 2 * k_dim), lambda i: (0, 0)),
            pl.BlockSpec(memory_space=pltpu.SMEM),
        ],
        out_specs=[
            pl.BlockSpec((1, RB), lambda i: (0, i)),
            pl.BlockSpec((num_models, RB), lambda i: (0, i)),
        ],
        out_shape=[
            jax.ShapeDtypeStruct((1, b), jnp.float32),
            jax.ShapeDtypeStruct((num_models, b), jnp.float32),
        ],
    )(pu, qi, rst, theta, bias)


@jax.jit
def _pla(u_idx, i_idx, r_s, p, q, theta, bias):
    u32 = u_idx.astype(jnp.int32)
    i32 = i_idx.astype(jnp.int32)
    pu, qi = _gather_sc(u32, i32, p, q)
    rhat2, alt = _dense_tc(pu, qi, r_s.T, theta, bias)
    return rhat2.reshape(r_s.shape[0]), alt.T


def kernel(u_idx, i_idx, r_s, P, Q, theta, bias):
    r_hat, alphas = _pla(u_idx, i_idx, r_s, P, Q, theta, bias)
    return (r_hat, alphas, r_s)
